# Initial kernel scaffold; baseline (speedup 1.0000x reference)
import jax
import jax.numpy as jnp
from jax.experimental import pallas as pl


def kernel(x, weights):
    raise NotImplementedError("write your pallas kernel here")


# speedup vs baseline: 21.8066x; 21.8066x over previous
import jax
import jax.numpy as jnp
from jax.experimental import pallas as pl


def _cim_kernel(x_ref, w_ref, o_ref):
    x = x_ref[...]
    w = w_ref[...]
    xn = jnp.sum(x * x, axis=1, keepdims=True)
    wn = jnp.sum(w * w, axis=1)[None, :]
    dot = jax.lax.dot_general(
        x, w, (((1,), (1,)), ((), ())),
        preferred_element_type=jnp.float32,
        precision=jax.lax.Precision.HIGHEST,
    )
    sq = jnp.maximum(xn + wn - 2.0 * dot, 0.0)
    o_ref[...] = jnp.sqrt(1.0 - jnp.exp(sq * -0.5) + 1e-8)


def kernel(x, weights):
    b, d = x.shape
    r, c, _ = weights.shape
    w2 = weights.reshape(r * c, d)
    out = pl.pallas_call(
        _cim_kernel,
        out_shape=jax.ShapeDtypeStruct((b, r * c), jnp.float32),
    )(x, w2)
    return out.reshape(b, r, c)


# speedup vs baseline: 23.5145x; 1.0783x over previous
import jax
import jax.numpy as jnp
from jax.experimental import pallas as pl


def _cim_kernel(x_ref, w_ref, o_ref):
    x = x_ref[...]
    w = w_ref[...]
    xn = jnp.sum(x * x, axis=1, keepdims=True)
    wn = jnp.sum(w * w, axis=1)[None, :]
    dot = jax.lax.dot_general(
        x, w, (((1,), (1,)), ((), ())),
        preferred_element_type=jnp.float32,
    )
    sq = jnp.maximum(xn + wn - 2.0 * dot, 0.0)
    o_ref[...] = jnp.sqrt(1.0 - jnp.exp(sq * -0.5) + 1e-8)


def kernel(x, weights):
    b, d = x.shape
    r, c, _ = weights.shape
    n = r * c
    w2 = weights.reshape(n, d)
    bm = 128
    out = pl.pallas_call(
        _cim_kernel,
        grid=(b // bm,),
        in_specs=[
            pl.BlockSpec((bm, d), lambda i: (i, 0)),
            pl.BlockSpec((n, d), lambda i: (0, 0)),
        ],
        out_specs=pl.BlockSpec((bm, n), lambda i: (i, 0)),
        out_shape=jax.ShapeDtypeStruct((b, n), jnp.float32),
    )(x, w2)
    return out.reshape(b, r, c)
